# baseline (speedup 1.0000x reference)
import functools

import jax
import jax.numpy as jnp
from jax import lax
from jax.experimental import pallas as pl
from jax.experimental.pallas import tpu as pltpu
from jax.experimental.pallas import tpu_sc as plsc

_N = 10000
_F = 128
_G = 64
_NC = 2
_NS = 16
_CL = 64
_CH = 160
_IB = 32
_NBK = _CH // _IB
_EPAD = _NC * _NS * _CH * _CL
_NACC = 10112
_STRIPE = _NACC // _NS
_RB = 1000
_NBLK = _N // _RB
_ZR = 40


def _make_agg():
    mesh = plsc.VectorSubcoreMesh(core_axis_name="c", subcore_axis_name="s")

    @functools.partial(
        pl.kernel,
        mesh=mesh,
        out_type=jax.ShapeDtypeStruct((_NC, _NACC, _F), jnp.float32),
        scratch_types=[
            pltpu.VMEM((2, _IB, _CL), jnp.int32),
            pltpu.VMEM((2, _IB, _CL), jnp.int32),
            pltpu.VMEM((4, _CL, _F), jnp.float32),
            pltpu.VMEM_SHARED((_NACC, _F), jnp.float32),
            pltpu.SemaphoreType.DMA,
            pltpu.SemaphoreType.DMA,
            pltpu.SemaphoreType.DMA,
            pltpu.SemaphoreType.DMA,
            pltpu.SemaphoreType.DMA,
            pltpu.SemaphoreType.DMA,
            pltpu.SemaphoreType.DMA,
            pltpu.SemaphoreType.DMA,
            pltpu.SemaphoreType.DMA,
        ],
    )
    def agg(h_hbm, src_hbm, dst_hbm, out_hbm, src_v, dst_v, rows_v, acc_sh,
            g0, g1, g2, g3, s0, s1, s2, s3, isem):
        gsems = (g0, g1, g2, g3)
        ssems = (s0, s1, s2, s3)
        c = lax.axis_index("c")
        s = lax.axis_index("s")

        def wait_gather(kb, j, b):
            pltpu.make_async_copy(h_hbm.at[src_v.at[kb, j]],
                                  rows_v.at[b], gsems[b]).wait()

        def start_gather(kb, j, b):
            pltpu.async_copy(h_hbm.at[src_v.at[kb, j]], rows_v.at[b], gsems[b])

        def start_scatter(kb, j, b):
            pltpu.async_copy(rows_v.at[b], acc_sh.at[dst_v.at[kb, j]],
                             ssems[b], add=True)

        def wait_scatter(kb, j, b):
            pltpu.make_async_copy(rows_v.at[b],
                                  acc_sh.at[dst_v.at[kb, j]], ssems[b]).wait()

        pltpu.sync_copy(src_hbm.at[c, s, pl.ds(0, _IB)], src_v.at[0])
        pltpu.sync_copy(dst_hbm.at[c, s, pl.ds(0, _IB)], dst_v.at[0])
        start_gather(0, 0, 0)
        start_gather(0, 1, 1)

        def zrow(r, carry):
            for q in range(_F // 16):
                rows_v[2, r, pl.ds(q * 16, 16)] = jnp.zeros((16,), jnp.float32)
            return carry

        lax.fori_loop(0, _CL, zrow, 0)
        zb = rows_v.at[2]

        @pl.when(jnp.logical_and(c == 0, s < _NS - 1))
        def _():
            pltpu.sync_copy(h_hbm.at[pl.ds(s * _STRIPE, _STRIPE)],
                            acc_sh.at[pl.ds(s * _STRIPE, _STRIPE)])

        @pl.when(jnp.logical_and(c == 0, s == _NS - 1))
        def _():
            pltpu.sync_copy(h_hbm.at[pl.ds(s * _STRIPE, _N - s * _STRIPE)],
                            acc_sh.at[pl.ds(s * _STRIPE, _N - s * _STRIPE)])
            pltpu.sync_copy(zb, acc_sh.at[pl.ds(_N, _CL)])
            pltpu.sync_copy(zb.at[pl.ds(0, _NACC - _N - _CL)],
                            acc_sh.at[pl.ds(_N + _CL, _NACC - _N - _CL)])

        @pl.when(c == 1)
        def _():
            def zcp(q, carry):
                pltpu.sync_copy(zb, acc_sh.at[pl.ds(s * _STRIPE + q * _CL, _CL)])
                return carry

            lax.fori_loop(0, _STRIPE // _CL, zcp, 0)
            pltpu.sync_copy(zb.at[pl.ds(0, _STRIPE % _CL)],
                            acc_sh.at[pl.ds(s * _STRIPE + _STRIPE - _STRIPE % _CL,
                                            _STRIPE % _CL)])

        plsc.subcore_barrier()

        for k in range(_NBK):
            kb, kbn = k % 2, (k + 1) % 2
            nxt = k + 1 < _NBK
            if nxt:
                pltpu.async_copy(src_hbm.at[c, s, pl.ds((k + 1) * _IB, _IB)],
                                 src_v.at[kbn], isem)
                pltpu.async_copy(dst_hbm.at[c, s, pl.ds((k + 1) * _IB, _IB)],
                                 dst_v.at[kbn], isem)

            if k == 0:
                for b in range(4):
                    b2 = (b + 2) % 4
                    wait_gather(0, b, b)
                    start_scatter(0, b, b)
                    if b >= 2:
                        wait_scatter(0, b, b2)
                    start_gather(0, b + 2, b2)
                glo = 1
            else:
                glo = 0

            def body(g4, carry, kb=kb):
                for b in range(4):
                    j = 4 * g4 + b
                    b2 = (b + 2) % 4
                    wait_gather(kb, j, b)
                    start_scatter(kb, j, b)
                    wait_scatter(kb, j, b2)
                    start_gather(kb, j + 2, b2)
                return carry

            lax.fori_loop(glo, _IB // 4 - 1, body, 0)

            if nxt:
                pltpu.make_async_copy(src_hbm.at[c, s, pl.ds((k + 1) * _IB, _IB)],
                                      src_v.at[kbn], isem).wait()
                pltpu.make_async_copy(dst_hbm.at[c, s, pl.ds((k + 1) * _IB, _IB)],
                                      dst_v.at[kbn], isem).wait()
            for b in range(4):
                j = _IB - 4 + b
                b2 = (b + 2) % 4
                wait_gather(kb, j, b)
                start_scatter(kb, j, b)
                if b < 2:
                    wait_scatter(kb, j, b2)
                    start_gather(kb, j + 2, b2)
                elif nxt:
                    wait_scatter(kb, j, b2)
                    start_gather(kbn, j + 2 - _IB, b2)

        for b in range(4):
            wait_scatter((_NBK - 1) % 2, _IB - 4 + b, b)
        plsc.subcore_barrier()
        pltpu.sync_copy(acc_sh.at[pl.ds(s * _STRIPE, _STRIPE)],
                        out_hbm.at[c, pl.ds(s * _STRIPE, _STRIPE)])

    return agg


_agg_cache = []


def _agg(h, src3, dst3):
    if not _agg_cache:
        _agg_cache.append(_make_agg())
    return _agg_cache[0](h, src3, dst3)


def _mlpbn_body(p_ref, wa_ref, ba_ref, wb_ref, bb_ref, g_ref, b_ref,
                o_ref, st_ref, y_sc):
    ph = pl.program_id(0)
    i = pl.program_id(1)

    @pl.when(ph == 0)
    def _():
        h = p_ref[0] + p_ref[1]
        a = jnp.maximum(
            jnp.dot(h, wa_ref[...], preferred_element_type=jnp.float32)
            + ba_ref[...], 0.0)
        y = jnp.dot(a, wb_ref[...], preferred_element_type=jnp.float32) + bb_ref[...]
        y_sc[i] = y

        @pl.when(i == 0)
        def _():
            st_ref[...] = jnp.zeros_like(st_ref)

        st_ref[0:1, :] += jnp.sum(y, axis=0, keepdims=True)
        st_ref[1:2, :] += jnp.sum(y * y, axis=0, keepdims=True)

    @pl.when(ph == 1)
    def _():
        y = y_sc[i]
        mu = st_ref[0:1, :] * (1.0 / _N)
        var = st_ref[1:2, :] * (1.0 / _N) - mu * mu
        scale = g_ref[...] * lax.rsqrt(var + 1e-5)
        o_ref[...] = jnp.maximum((y - mu) * scale + b_ref[...], 0.0)


def _mlpbn(partials, wa, ba, wb, bb, gamma, beta):
    h, _ = pl.pallas_call(
        _mlpbn_body,
        grid=(2, _NBLK),
        in_specs=[
            pl.BlockSpec((_NC, _RB, _F), lambda p, i: (0, (1 - p) * i, 0)),
            pl.BlockSpec((_F, _F), lambda p, i: (0, 0)),
            pl.BlockSpec((1, _F), lambda p, i: (0, 0)),
            pl.BlockSpec((_F, _F), lambda p, i: (0, 0)),
            pl.BlockSpec((1, _F), lambda p, i: (0, 0)),
            pl.BlockSpec((1, _F), lambda p, i: (0, 0)),
            pl.BlockSpec((1, _F), lambda p, i: (0, 0)),
        ],
        out_specs=[
            pl.BlockSpec((_RB, _F), lambda p, i: (p * i, 0)),
            pl.BlockSpec((8, _F), lambda p, i: (0, 0)),
        ],
        out_shape=[
            jax.ShapeDtypeStruct((_N, _F), jnp.float32),
            jax.ShapeDtypeStruct((8, _F), jnp.float32),
        ],
        scratch_shapes=[pltpu.VMEM((_NBLK, _RB, _F), jnp.float32)],
    )(partials, wa, ba.reshape(1, _F), wb, bb.reshape(1, _F),
      gamma.reshape(1, _F), beta.reshape(1, _F))
    return h


def _mlphead_body(p_ref, wa_ref, ba_ref, wb_ref, bb_ref, g_ref, b_ref,
                  batch_ref, tf_ref, pw_ref, pb_ref, rw1_ref, rb1_ref, rw2_ref,
                  out_ref, st_ref, y_sc):
    ph = pl.program_id(0)
    i = pl.program_id(1)

    @pl.when(ph == 0)
    def _():
        h = p_ref[0] + p_ref[1]
        a = jnp.maximum(
            jnp.dot(h, wa_ref[...], preferred_element_type=jnp.float32)
            + ba_ref[...], 0.0)
        y = jnp.dot(a, wb_ref[...], preferred_element_type=jnp.float32) + bb_ref[...]
        y_sc[i] = y

        @pl.when(i == 0)
        def _():
            st_ref[...] = jnp.zeros_like(st_ref)

        st_ref[0:1, :] += jnp.sum(y, axis=0, keepdims=True)
        st_ref[1:2, :] += jnp.sum(y * y, axis=0, keepdims=True)

    @pl.when(ph == 1)
    def _():
        y = y_sc[i]
        mu = st_ref[0:1, :] * (1.0 / _N)
        var = st_ref[1:2, :] * (1.0 / _N) - mu * mu
        scale = g_ref[...] * lax.rsqrt(var + 1e-5)
        h = jnp.maximum((y - mu) * scale + b_ref[...], 0.0)

        onehot = (batch_ref[...] == lax.broadcasted_iota(jnp.int32, (1, _G), 1)
                  ).astype(jnp.float32)
        contrib = lax.dot_general(onehot, h, (((0,), (0,)), ((), ())),
                                  preferred_element_type=jnp.float32)

        @pl.when(i == 0)
        def _():
            out_ref[...] = jnp.zeros_like(out_ref)

        out_ref[...] += contrib

        @pl.when(i == pl.num_programs(1) - 1)
        def _():
            emb = out_ref[...]
            temb = jnp.maximum(
                jnp.dot(tf_ref[...], pw_ref[...], preferred_element_type=jnp.float32)
                + pb_ref[...], 0.0)
            r1 = jnp.maximum(
                jnp.dot(emb, rw1_ref[0:_F, :], preferred_element_type=jnp.float32)
                + jnp.dot(temb, rw1_ref[_F:2 * _F, :],
                          preferred_element_type=jnp.float32)
                + rb1_ref[...], 0.0)
            r2 = jnp.sum(r1 * rw2_ref[...], axis=1, keepdims=True)
            out_ref[...] = jnp.broadcast_to(r2, (_G, _F))


def _mlphead(partials, wa, ba, wb, bb, gamma, beta, batch2d, target_feat,
             proj_w, proj_b, reg_w1, reg_b1, reg_w2):
    out, _ = pl.pallas_call(
        _mlphead_body,
        grid=(2, _NBLK),
        in_specs=[
            pl.BlockSpec((_NC, _RB, _F), lambda p, i: (0, (1 - p) * i, 0)),
            pl.BlockSpec((_F, _F), lambda p, i: (0, 0)),
            pl.BlockSpec((1, _F), lambda p, i: (0, 0)),
            pl.BlockSpec((_F, _F), lambda p, i: (0, 0)),
            pl.BlockSpec((1, _F), lambda p, i: (0, 0)),
            pl.BlockSpec((1, _F), lambda p, i: (0, 0)),
            pl.BlockSpec((1, _F), lambda p, i: (0, 0)),
            pl.BlockSpec((_RB, 1), lambda p, i: (p * i, 0)),
            pl.BlockSpec((_G, _F), lambda p, i: (0, 0)),
            pl.BlockSpec((_F, _F), lambda p, i: (0, 0)),
            pl.BlockSpec((1, _F), lambda p, i: (0, 0)),
            pl.BlockSpec((2 * _F, _F), lambda p, i: (0, 0)),
            pl.BlockSpec((1, _F), lambda p, i: (0, 0)),
            pl.BlockSpec((1, _F), lambda p, i: (0, 0)),
        ],
        out_specs=[
            pl.BlockSpec((_G, _F), lambda p, i: (0, 0)),
            pl.BlockSpec((8, _F), lambda p, i: (0, 0)),
        ],
        out_shape=[
            jax.ShapeDtypeStruct((_G, _F), jnp.float32),
            jax.ShapeDtypeStruct((8, _F), jnp.float32),
        ],
        scratch_shapes=[pltpu.VMEM((_NBLK, _RB, _F), jnp.float32)],
    )(partials, wa, ba.reshape(1, _F), wb, bb.reshape(1, _F),
      gamma.reshape(1, _F), beta.reshape(1, _F), batch2d, target_feat,
      proj_w, proj_b.reshape(1, _F), reg_w1, reg_b1.reshape(1, _F),
      reg_w2.reshape(1, _F))
    return out


def kernel(x, edge_index, batch, target_feat,
           w1a, b1a, w1b, b1b, gamma1, beta1,
           w2a, b2a, w2b, b2b, gamma2, beta2,
           w3a, b3a, w3b, b3b, gamma3, beta3,
           proj_w, proj_b, reg_w1, reg_b1, reg_w2, reg_b2):
    e = edge_index.shape[1]
    pad = _EPAD - e
    pad_src = (jnp.arange(pad, dtype=jnp.int32) % _N)
    pad_dst = _N + (jnp.arange(pad, dtype=jnp.int32) % (_NACC - _N))
    src3 = jnp.concatenate([edge_index[0], pad_src]).reshape(_NC, _NS, _CH, _CL)
    dst3 = jnp.concatenate([edge_index[1], pad_dst]).reshape(_NC, _NS, _CH, _CL)
    batch2d = batch.reshape(_N, 1)

    h = x
    layers = [(w1a, b1a, w1b, b1b, gamma1, beta1),
              (w2a, b2a, w2b, b2b, gamma2, beta2),
              (w3a, b3a, w3b, b3b, gamma3, beta3)]
    out128 = None
    for li, (wa, ba, wb, bb, g, be) in enumerate(layers):
        partials = _agg(h, src3, dst3)
        if li < 2:
            h = _mlpbn(partials, wa, ba, wb, bb, g, be)
        else:
            out128 = _mlphead(partials, wa, ba, wb, bb, g, be, batch2d,
                              target_feat, proj_w, proj_b, reg_w1, reg_b1, reg_w2)
    return out128[:, 0] + reg_b2[0]


# speedup vs baseline: 1.1129x; 1.1129x over previous
import functools

import jax
import jax.numpy as jnp
from jax import lax
from jax.experimental import pallas as pl
from jax.experimental.pallas import tpu as pltpu
from jax.experimental.pallas import tpu_sc as plsc

_N = 10000
_F = 128
_G = 64
_NC = 2
_NS = 16
_CL = 128
_CH = 80
_IB = 16
_NBK = _CH // _IB
_EPAD = _NC * _NS * _CH * _CL
_NACC = 10240
_STRIPE = _NACC // _NS
_RB = 1000
_NBLK = _N // _RB
_ZR = 40


def _make_agg():
    mesh = plsc.VectorSubcoreMesh(core_axis_name="c", subcore_axis_name="s")

    @functools.partial(
        pl.kernel,
        mesh=mesh,
        out_type=jax.ShapeDtypeStruct((_NC, _NACC, _F), jnp.float32),
        scratch_types=[
            pltpu.VMEM((2, _IB, _CL), jnp.int32),
            pltpu.VMEM((2, _IB, _CL), jnp.int32),
            pltpu.VMEM((2, _CL, _F), jnp.float32),
            pltpu.VMEM((_ZR, _F), jnp.float32),
            pltpu.VMEM_SHARED((_NACC, _F), jnp.float32),
            pltpu.SemaphoreType.DMA,
            pltpu.SemaphoreType.DMA,
            pltpu.SemaphoreType.DMA,
        ],
    )
    def agg(h_hbm, src_hbm, dst_hbm, out_hbm, src_v, dst_v, rows_v, zb_v, acc_sh,
            gsem0, gsem1, isem):
        c = lax.axis_index("c")
        s = lax.axis_index("s")
        pltpu.sync_copy(src_hbm.at[c, s, pl.ds(0, _IB)], src_v.at[0])
        pltpu.sync_copy(dst_hbm.at[c, s, pl.ds(0, _IB)], dst_v.at[0])
        pltpu.async_copy(h_hbm.at[src_v.at[0, 0]], rows_v.at[0], gsem0)
        pltpu.async_copy(h_hbm.at[src_v.at[0, 1]], rows_v.at[1], gsem1)

        def zrow(r, carry):
            for q in range(_F // 16):
                zb_v[r, pl.ds(q * 16, 16)] = jnp.zeros((16,), jnp.float32)
            return carry

        lax.fori_loop(0, _ZR, zrow, 0)

        @pl.when(jnp.logical_and(c == 0, s < _NS - 1))
        def _():
            pltpu.sync_copy(h_hbm.at[pl.ds(s * _STRIPE, _STRIPE)],
                            acc_sh.at[pl.ds(s * _STRIPE, _STRIPE)])

        @pl.when(jnp.logical_and(c == 0, s == _NS - 1))
        def _():
            pltpu.sync_copy(h_hbm.at[pl.ds(s * _STRIPE, _N - s * _STRIPE)],
                            acc_sh.at[pl.ds(s * _STRIPE, _N - s * _STRIPE)])
            for q in range((_NACC - _N) // _ZR):
                pltpu.sync_copy(zb_v, acc_sh.at[pl.ds(_N + q * _ZR, _ZR)])

        @pl.when(c == 1)
        def _():
            def zcp(q, carry):
                pltpu.sync_copy(zb_v, acc_sh.at[pl.ds(s * _STRIPE + q * _ZR, _ZR)])
                return carry

            lax.fori_loop(0, _STRIPE // _ZR, zcp, 0)

        plsc.subcore_barrier()

        for k in range(_NBK):
            kb, kbn = k % 2, (k + 1) % 2
            nxt = k + 1 < _NBK
            if nxt:
                pltpu.async_copy(src_hbm.at[c, s, pl.ds((k + 1) * _IB, _IB)],
                                 src_v.at[kbn], isem)
                pltpu.async_copy(dst_hbm.at[c, s, pl.ds((k + 1) * _IB, _IB)],
                                 dst_v.at[kbn], isem)

            def body(g, carry, kb=kb):
                for b, sem in ((0, gsem0), (1, gsem1)):
                    j = 2 * g + b
                    pltpu.make_async_copy(h_hbm.at[src_v.at[kb, j]],
                                          rows_v.at[b], sem).wait()
                    pltpu.sync_copy(rows_v.at[b], acc_sh.at[dst_v.at[kb, j]], add=True)
                    pltpu.async_copy(h_hbm.at[src_v.at[kb, j + 2]], rows_v.at[b], sem)
                return carry

            lax.fori_loop(0, _IB // 2 - 1, body, 0)

            if nxt:
                pltpu.make_async_copy(src_hbm.at[c, s, pl.ds((k + 1) * _IB, _IB)],
                                      src_v.at[kbn], isem).wait()
                pltpu.make_async_copy(dst_hbm.at[c, s, pl.ds((k + 1) * _IB, _IB)],
                                      dst_v.at[kbn], isem).wait()
            for b, sem in ((0, gsem0), (1, gsem1)):
                j = _IB - 2 + b
                pltpu.make_async_copy(h_hbm.at[src_v.at[kb, j]], rows_v.at[b], sem).wait()
                pltpu.sync_copy(rows_v.at[b], acc_sh.at[dst_v.at[kb, j]], add=True)
                if nxt:
                    pltpu.async_copy(h_hbm.at[src_v.at[kbn, b]], rows_v.at[b], sem)

        plsc.subcore_barrier()
        pltpu.sync_copy(acc_sh.at[pl.ds(s * _STRIPE, _STRIPE)],
                        out_hbm.at[c, pl.ds(s * _STRIPE, _STRIPE)])

    return agg


_agg_cache = []


def _agg(h, src3, dst3):
    if not _agg_cache:
        _agg_cache.append(_make_agg())
    return _agg_cache[0](h, src3, dst3)


def _mlpbn_body(p_ref, wa_ref, ba_ref, wb_ref, bb_ref, g_ref, b_ref,
                o_ref, st_ref, y_sc):
    ph = pl.program_id(0)
    i = pl.program_id(1)

    @pl.when(ph == 0)
    def _():
        h = p_ref[0] + p_ref[1]
        a = jnp.maximum(
            jnp.dot(h, wa_ref[...], preferred_element_type=jnp.float32)
            + ba_ref[...], 0.0)
        y = jnp.dot(a, wb_ref[...], preferred_element_type=jnp.float32) + bb_ref[...]
        y_sc[i] = y

        @pl.when(i == 0)
        def _():
            st_ref[...] = jnp.zeros_like(st_ref)

        st_ref[0:1, :] += jnp.sum(y, axis=0, keepdims=True)
        st_ref[1:2, :] += jnp.sum(y * y, axis=0, keepdims=True)

    @pl.when(ph == 1)
    def _():
        y = y_sc[i]
        mu = st_ref[0:1, :] * (1.0 / _N)
        var = st_ref[1:2, :] * (1.0 / _N) - mu * mu
        scale = g_ref[...] * lax.rsqrt(var + 1e-5)
        o_ref[...] = jnp.maximum((y - mu) * scale + b_ref[...], 0.0)


def _mlpbn(partials, wa, ba, wb, bb, gamma, beta):
    h, _ = pl.pallas_call(
        _mlpbn_body,
        grid=(2, _NBLK),
        in_specs=[
            pl.BlockSpec((_NC, _RB, _F), lambda p, i: (0, (1 - p) * i, 0)),
            pl.BlockSpec((_F, _F), lambda p, i: (0, 0)),
            pl.BlockSpec((1, _F), lambda p, i: (0, 0)),
            pl.BlockSpec((_F, _F), lambda p, i: (0, 0)),
            pl.BlockSpec((1, _F), lambda p, i: (0, 0)),
            pl.BlockSpec((1, _F), lambda p, i: (0, 0)),
            pl.BlockSpec((1, _F), lambda p, i: (0, 0)),
        ],
        out_specs=[
            pl.BlockSpec((_RB, _F), lambda p, i: (p * i, 0)),
            pl.BlockSpec((8, _F), lambda p, i: (0, 0)),
        ],
        out_shape=[
            jax.ShapeDtypeStruct((_N, _F), jnp.float32),
            jax.ShapeDtypeStruct((8, _F), jnp.float32),
        ],
        scratch_shapes=[pltpu.VMEM((_NBLK, _RB, _F), jnp.float32)],
    )(partials, wa, ba.reshape(1, _F), wb, bb.reshape(1, _F),
      gamma.reshape(1, _F), beta.reshape(1, _F))
    return h


def _mlphead_body(p_ref, wa_ref, ba_ref, wb_ref, bb_ref, g_ref, b_ref,
                  batch_ref, tf_ref, pw_ref, pb_ref, rw1_ref, rb1_ref, rw2_ref,
                  out_ref, st_ref, y_sc):
    ph = pl.program_id(0)
    i = pl.program_id(1)

    @pl.when(ph == 0)
    def _():
        h = p_ref[0] + p_ref[1]
        a = jnp.maximum(
            jnp.dot(h, wa_ref[...], preferred_element_type=jnp.float32)
            + ba_ref[...], 0.0)
        y = jnp.dot(a, wb_ref[...], preferred_element_type=jnp.float32) + bb_ref[...]
        y_sc[i] = y

        @pl.when(i == 0)
        def _():
            st_ref[...] = jnp.zeros_like(st_ref)

        st_ref[0:1, :] += jnp.sum(y, axis=0, keepdims=True)
        st_ref[1:2, :] += jnp.sum(y * y, axis=0, keepdims=True)

    @pl.when(ph == 1)
    def _():
        y = y_sc[i]
        mu = st_ref[0:1, :] * (1.0 / _N)
        var = st_ref[1:2, :] * (1.0 / _N) - mu * mu
        scale = g_ref[...] * lax.rsqrt(var + 1e-5)
        h = jnp.maximum((y - mu) * scale + b_ref[...], 0.0)

        onehot = (batch_ref[...] == lax.broadcasted_iota(jnp.int32, (1, _G), 1)
                  ).astype(jnp.float32)
        contrib = lax.dot_general(onehot, h, (((0,), (0,)), ((), ())),
                                  preferred_element_type=jnp.float32)

        @pl.when(i == 0)
        def _():
            out_ref[...] = jnp.zeros_like(out_ref)

        out_ref[...] += contrib

        @pl.when(i == pl.num_programs(1) - 1)
        def _():
            emb = out_ref[...]
            temb = jnp.maximum(
                jnp.dot(tf_ref[...], pw_ref[...], preferred_element_type=jnp.float32)
                + pb_ref[...], 0.0)
            r1 = jnp.maximum(
                jnp.dot(emb, rw1_ref[0:_F, :], preferred_element_type=jnp.float32)
                + jnp.dot(temb, rw1_ref[_F:2 * _F, :],
                          preferred_element_type=jnp.float32)
                + rb1_ref[...], 0.0)
            r2 = jnp.sum(r1 * rw2_ref[...], axis=1, keepdims=True)
            out_ref[...] = jnp.broadcast_to(r2, (_G, _F))


def _mlphead(partials, wa, ba, wb, bb, gamma, beta, batch2d, target_feat,
             proj_w, proj_b, reg_w1, reg_b1, reg_w2):
    out, _ = pl.pallas_call(
        _mlphead_body,
        grid=(2, _NBLK),
        in_specs=[
            pl.BlockSpec((_NC, _RB, _F), lambda p, i: (0, (1 - p) * i, 0)),
            pl.BlockSpec((_F, _F), lambda p, i: (0, 0)),
            pl.BlockSpec((1, _F), lambda p, i: (0, 0)),
            pl.BlockSpec((_F, _F), lambda p, i: (0, 0)),
            pl.BlockSpec((1, _F), lambda p, i: (0, 0)),
            pl.BlockSpec((1, _F), lambda p, i: (0, 0)),
            pl.BlockSpec((1, _F), lambda p, i: (0, 0)),
            pl.BlockSpec((_RB, 1), lambda p, i: (p * i, 0)),
            pl.BlockSpec((_G, _F), lambda p, i: (0, 0)),
            pl.BlockSpec((_F, _F), lambda p, i: (0, 0)),
            pl.BlockSpec((1, _F), lambda p, i: (0, 0)),
            pl.BlockSpec((2 * _F, _F), lambda p, i: (0, 0)),
            pl.BlockSpec((1, _F), lambda p, i: (0, 0)),
            pl.BlockSpec((1, _F), lambda p, i: (0, 0)),
        ],
        out_specs=[
            pl.BlockSpec((_G, _F), lambda p, i: (0, 0)),
            pl.BlockSpec((8, _F), lambda p, i: (0, 0)),
        ],
        out_shape=[
            jax.ShapeDtypeStruct((_G, _F), jnp.float32),
            jax.ShapeDtypeStruct((8, _F), jnp.float32),
        ],
        scratch_shapes=[pltpu.VMEM((_NBLK, _RB, _F), jnp.float32)],
    )(partials, wa, ba.reshape(1, _F), wb, bb.reshape(1, _F),
      gamma.reshape(1, _F), beta.reshape(1, _F), batch2d, target_feat,
      proj_w, proj_b.reshape(1, _F), reg_w1, reg_b1.reshape(1, _F),
      reg_w2.reshape(1, _F))
    return out


def kernel(x, edge_index, batch, target_feat,
           w1a, b1a, w1b, b1b, gamma1, beta1,
           w2a, b2a, w2b, b2b, gamma2, beta2,
           w3a, b3a, w3b, b3b, gamma3, beta3,
           proj_w, proj_b, reg_w1, reg_b1, reg_w2, reg_b2):
    e = edge_index.shape[1]
    pad = _EPAD - e
    pad_src = (jnp.arange(pad, dtype=jnp.int32) % _N)
    pad_dst = _N + (jnp.arange(pad, dtype=jnp.int32) % (_NACC - _N))
    src3 = jnp.concatenate([edge_index[0], pad_src]).reshape(_NC, _NS, _CH, _CL)
    dst3 = jnp.concatenate([edge_index[1], pad_dst]).reshape(_NC, _NS, _CH, _CL)
    batch2d = batch.reshape(_N, 1)

    h = x
    layers = [(w1a, b1a, w1b, b1b, gamma1, beta1),
              (w2a, b2a, w2b, b2b, gamma2, beta2),
              (w3a, b3a, w3b, b3b, gamma3, beta3)]
    out128 = None
    for li, (wa, ba, wb, bb, g, be) in enumerate(layers):
        partials = _agg(h, src3, dst3)
        if li < 2:
            h = _mlpbn(partials, wa, ba, wb, bb, g, be)
        else:
            out128 = _mlphead(partials, wa, ba, wb, bb, g, be, batch2d,
                              target_feat, proj_w, proj_b, reg_w1, reg_b1, reg_w2)
    return out128[:, 0] + reg_b2[0]
